# baseline (speedup 1.0000x reference)
import functools

import jax
import jax.numpy as jnp
from jax import lax
from jax.experimental import pallas as pl
from jax.experimental.pallas import tpu as pltpu
from jax.experimental.pallas import tpu_sc as plsc

_B = 4096
_DG, _DC = 512, 1024
_HID = 512
_GH = 128
_NC = 5
_K = 8
_ALPHA = 0.5

_R1 = 512
_R4 = 512

_NW = 32
_BPW = _B // _NW
_CH = 4


def _mm(x, w):
    return lax.dot_general(x, w, (((1,), (1,)), ((), ())),
                           preferred_element_type=jnp.float32)


def _p1_body(x_ref, xwin, lngw, lngb, lncw, lncb, pgw, pgb, pcw, pcb,
             rpw, rpb, rsw, rsb, g1w, g1b, g2w, g2b,
             fused_ref, aux_ref, idx_ref, w_ref, acc_ref, xt):
    i = pl.program_id(0)
    nb = pl.num_programs(0)

    @pl.when(i == 0)
    def _():
        acc_ref[0] = 0.0
        acc_ref[1] = 0.0
        acc_ref[2] = 0.0
        xt[...] = jnp.transpose(xwin[...])

    gr = x_ref[:, :_DG]
    cr = x_ref[:, _DG:_DG + _DC]
    gm = jnp.mean(gr, axis=1, keepdims=True)
    gv = jnp.mean((gr - gm) ** 2, axis=1, keepdims=True)
    gln = (gr - gm) / jnp.sqrt(gv + 1e-5) * lngw[...] + lngb[...]
    cm = jnp.mean(cr, axis=1, keepdims=True)
    cv = jnp.mean((cr - cm) ** 2, axis=1, keepdims=True)
    cln = (cr - cm) / jnp.sqrt(cv + 1e-5) * lncw[...] + lncb[...]

    g = _mm(gln, pgw[...]) + pgb[...]
    c = _mm(cln, pcw[...]) + pcb[...]
    rp = _mm(g, rpw[...]) + rpb[...]
    rs = _mm(g, rsw[...]) + rsb[...]

    concat = jnp.concatenate([rp, c], axis=1)
    gh = jnp.maximum(_mm(concat, g1w[...]) + g1b[...], 0.0)
    gl0 = jnp.sum(gh * g2w[0:1, :], axis=1, keepdims=True) + g2b[:, 0:1]
    gl1 = jnp.sum(gh * g2w[1:2, :], axis=1, keepdims=True) + g2b[:, 1:2]
    m = jnp.maximum(gl0, gl1)
    e0 = jnp.exp(gl0 - m)
    e1 = jnp.exp(gl1 - m)
    s = e0 + e1
    gp0 = e0 / s
    gp1 = e1 / s

    fused_ref[...] = jnp.concatenate([rp * gp0, c * gp1], axis=1)

    ent = -(gp0 * jnp.log(gp0 + 1e-8) + gp1 * jnp.log(gp1 + 1e-8))
    acc_ref[0] += jnp.sum(ent)

    rpn = rp / jnp.maximum(jnp.sqrt(jnp.sum(rp * rp, axis=1, keepdims=True)), 1e-12)
    rsn = rs / jnp.maximum(jnp.sqrt(jnp.sum(rs * rs, axis=1, keepdims=True)), 1e-12)
    acc_ref[1] += jnp.sum(jnp.abs(jnp.sum(rpn * rsn, axis=1)))

    acc_ref[2] += jnp.sum((rs - c) ** 2)

    @pl.when(i == nb - 1)
    def _():
        aux_ref[0, 0] = (0.01 * (acc_ref[0] / _B)
                         + 0.02 * (acc_ref[1] / _B)
                         + 0.01 * (acc_ref[2] / (_B * _HID)))

    xi = x_ref[:, _DG + _DC:_DG + _DC + 1]
    yi = x_ref[:, _DG + _DC + 1:_DG + _DC + 2]
    mi = x_ref[:, _DG + _DC + 4:_DG + _DC + 5]
    xj = xt[0:1, :]
    yj = xt[1:2, :]
    mj = xt[4:5, :]

    dx = xi - xj
    dy = yi - yj
    d2 = dx * dx + dy * dy

    _SENT = 4e18
    _GONE = 1e19

    rowid = i * _R1 + lax.broadcasted_iota(jnp.int32, (_R1, 1), 0)
    colid = lax.broadcasted_iota(jnp.int32, (_R1, _B), 1)
    eye = colid == rowid
    same = (mi == mj) & jnp.logical_not(eye)
    has = jnp.any(same, axis=1, keepdims=True)
    d2 = jnp.where(has, jnp.where(same, d2, _SENT), d2)
    d2 = jnp.where(eye, _SENT, d2)

    work = d2
    vals = []
    idxs = []
    for k in range(_K):
        mn = jnp.min(work, axis=1, keepdims=True)
        am = jnp.min(jnp.where(work == mn, colid, _B), axis=1, keepdims=True)
        vals.append(mn)
        idxs.append(am)
        if k < _K - 1:
            work = jnp.where(colid == am, _GONE, work)

    v8 = jnp.concatenate(vals, axis=1)
    idx_ref[...] = jnp.concatenate(idxs, axis=1)
    s8 = jnp.where(v8 >= _SENT * 0.5, -1e9, -jnp.sqrt(v8 + 1e-12))
    vm = jnp.max(s8, axis=1, keepdims=True)
    ev = jnp.exp(s8 - vm)
    w = ev / jnp.sum(ev, axis=1, keepdims=True)
    w_ref[...] = jnp.concatenate([w, jnp.zeros((_R1, 16 - _K), jnp.float32)],
                                 axis=1)


def _p4_body(agg_ref, fused_ref, g1w, g1b, g2w, g2b, c1w, c1b,
             bnw, bnb, bnm, bnv, c2w, c2b, out_ref):
    h1 = jnp.maximum(_mm(agg_ref[...], g1w[...]) + g1b[...], 0.0)
    gu = _mm(h1, g2w[...]) + g2b[...]
    ctx = fused_ref[...] + _ALPHA * gu
    h = _mm(ctx, c1w[...]) + c1b[...]
    h = (h - bnm[...]) / jnp.sqrt(bnv[...] + 1e-5) * bnw[...] + bnb[...]
    h = jnp.maximum(h, 0.0)
    out_ref[...] = _mm(h, c2w[...]) + c2b[...]


_CHK = _CH * _K
_NCH = _BPW // _CH


def _sc_body(fused_hbm, idx_hbm, w_hbm, out_hbm,
             idx_v, w0, w1, rows0, rows1, out0, out1, sem0, sem1):
    wid = lax.axis_index("s") * 2 + lax.axis_index("c")
    base = wid * _BPW

    pltpu.sync_copy(idx_hbm.at[pl.ds(base * _K, _BPW * _K)], idx_v)

    rows = (rows0, rows1)
    wbufs = (w0, w1)
    outs = (out0, out1)
    sems = (sem0, sem1)

    def gidx(ci):
        return idx_v.at[pl.ds(ci * _CHK, _CHK)]

    def wsrc(ci):
        return w_hbm.at[pl.ds(base + ci * _CH, _CH)]

    def stage(ci, b):
        pltpu.async_copy(wsrc(ci), wbufs[b], sems[b])
        pltpu.async_copy(fused_hbm.at[gidx(ci)], rows[b], sems[b])

    def compute(ci, b):
        pltpu.make_async_copy(wsrc(ci), wbufs[b], sems[b]).wait()
        pltpu.make_async_copy(fused_hbm.at[gidx(ci)], rows[b], sems[b]).wait()
        rv = rows[b]
        ov = outs[b]
        for r in range(_CH):
            wrow = wbufs[b][r]
            dn = lax.GatherDimensionNumbers(
                offset_dims=(), collapsed_slice_dims=(0,),
                start_index_map=(0,))
            wv = [lax.gather(wrow, jnp.full((16, 1), k, jnp.int32), dn, (1,),
                             mode=lax.GatherScatterMode.PROMISE_IN_BOUNDS)
                  for k in range(_K)]

            def colbody(cc, cr2):
                for j in range(4):
                    sl = pl.ds(cc * 64 + j * 16, 16)
                    acc = rv[r * _K, sl] * wv[0]
                    for k in range(1, _K):
                        acc = acc + rv[r * _K + k, sl] * wv[k]
                    ov[r, sl] = acc
                return cr2

            lax.fori_loop(0, _DC // 64, colbody, 0)
        pltpu.sync_copy(ov, out_hbm.at[pl.ds(base + ci * _CH, _CH)])

    stage(0, 0)

    def lbody(j, carry):
        ci0 = 2 * j
        stage(ci0 + 1, 1)
        compute(ci0, 0)
        stage(ci0 + 2, 0)
        compute(ci0 + 1, 1)
        return carry

    lax.fori_loop(0, _NCH // 2 - 1, lbody, 0)
    stage(_NCH - 1, 1)
    compute(_NCH - 2, 0)
    compute(_NCH - 1, 1)


def _sc_aggregate(fused, idx_flat, w_exp):
    mesh = plsc.VectorSubcoreMesh(core_axis_name="c", subcore_axis_name="s")
    return pl.kernel(
        _sc_body,
        out_type=jax.ShapeDtypeStruct((_B, _DC), jnp.float32),
        mesh=mesh,
        scratch_types=[
            pltpu.VMEM((_BPW * _K,), jnp.int32),
            pltpu.VMEM((_CH, 16), jnp.float32),
            pltpu.VMEM((_CH, 16), jnp.float32),
            pltpu.VMEM((_CHK, _DC), jnp.float32),
            pltpu.VMEM((_CHK, _DC), jnp.float32),
            pltpu.VMEM((_CH, _DC), jnp.float32),
            pltpu.VMEM((_CH, _DC), jnp.float32),
            pltpu.SemaphoreType.DMA,
            pltpu.SemaphoreType.DMA,
        ],
    )(fused, idx_flat, w_exp)


def kernel(x, ln_g_w, ln_g_b, ln_c_w, ln_c_b, proj_g_W, proj_g_b, proj_c_W,
           proj_c_b, relp_W, relp_b, rels_W, rels_b, gate1_W, gate1_b,
           gate2_W, gate2_b, gu1_W, gu1_b, gu2_W, gu2_b, cls1_W, cls1_b,
           bn_w, bn_b, bn_mean, bn_var, cls2_W, cls2_b):
    xw = x.shape[1]

    row = lambda v: v.reshape(1, -1)
    full = lambda shape: pl.BlockSpec(shape, lambda i: (0, 0))

    nb1 = _B // _R1
    fused, aux, idx8, w8 = pl.pallas_call(
        _p1_body,
        grid=(nb1,),
        in_specs=[
            pl.BlockSpec((_R1, xw), lambda i: (i, 0)),
            pl.BlockSpec((_B, 128), lambda i: (0, (_DG + _DC) // 128)),
            full((1, _DG)), full((1, _DG)), full((1, _DC)), full((1, _DC)),
            full((_HID, _DG)), full((1, _HID)),
            full((_HID, _DC)), full((1, _HID)),
            full((_HID, _HID)), full((1, _HID)),
            full((_HID, _HID)), full((1, _HID)),
            full((_GH, 2 * _HID)), full((1, _GH)),
            full((2, _GH)), full((1, 2)),
        ],
        out_specs=[
            pl.BlockSpec((_R1, _DC), lambda i: (i, 0)),
            pl.BlockSpec(memory_space=pltpu.SMEM),
            pl.BlockSpec((_R1, _K), lambda i: (i, 0)),
            pl.BlockSpec((_R1, 16), lambda i: (i, 0)),
        ],
        out_shape=[
            jax.ShapeDtypeStruct((_B, _DC), jnp.float32),
            jax.ShapeDtypeStruct((1, 1), jnp.float32),
            jax.ShapeDtypeStruct((_B, _K), jnp.int32),
            jax.ShapeDtypeStruct((_B, 16), jnp.float32),
        ],
        scratch_shapes=[pltpu.SMEM((3,), jnp.float32),
                        pltpu.VMEM((128, _B), jnp.float32)],
    )(x, x, row(ln_g_w), row(ln_g_b), row(ln_c_w), row(ln_c_b),
      proj_g_W, row(proj_g_b), proj_c_W, row(proj_c_b),
      relp_W, row(relp_b), rels_W, row(rels_b),
      gate1_W, row(gate1_b), gate2_W, row(gate2_b))

    agg = _sc_aggregate(fused, idx8.reshape(-1), w8)

    nb4 = _B // _R4
    logits = pl.pallas_call(
        _p4_body,
        grid=(nb4,),
        in_specs=[
            pl.BlockSpec((_R4, _DC), lambda i: (i, 0)),
            pl.BlockSpec((_R4, _DC), lambda i: (i, 0)),
            full((2 * _HID, 2 * _HID)), full((1, 2 * _HID)),
            full((2 * _HID, 2 * _HID)), full((1, 2 * _HID)),
            full((_HID, 2 * _HID)), full((1, _HID)),
            full((1, _HID)), full((1, _HID)), full((1, _HID)), full((1, _HID)),
            full((_NC, _HID)), full((1, _NC)),
        ],
        out_specs=pl.BlockSpec((_R4, _NC), lambda i: (i, 0)),
        out_shape=jax.ShapeDtypeStruct((_B, _NC), jnp.float32),
    )(agg, fused, gu1_W, row(gu1_b), gu2_W, row(gu2_b),
      cls1_W, row(cls1_b), row(bn_w), row(bn_b), row(bn_mean), row(bn_var),
      cls2_W, row(cls2_b))

    return logits, aux[0, 0]


# speedup vs baseline: 1.0404x; 1.0404x over previous
import functools

import jax
import jax.numpy as jnp
from jax import lax
from jax.experimental import pallas as pl
from jax.experimental.pallas import tpu as pltpu
from jax.experimental.pallas import tpu_sc as plsc

_B = 4096
_DG, _DC = 512, 1024
_HID = 512
_GH = 128
_NC = 5
_K = 8
_ALPHA = 0.5

_R1 = 512
_R4 = 512

_NW = 32
_BPW = _B // _NW
_CH = 4


def _mm(x, w):
    return lax.dot_general(x, w, (((1,), (1,)), ((), ())),
                           preferred_element_type=jnp.float32)


def _p1_body(gT, cTa, cTb, xjr, yjr, mjr, lngw, lngb, lncw, lncb,
             pgw, pgb, pcw, pcb,
             rpw, rpb, rsw, rsb, g1w, g1b, g2w, g2b,
             fused_ref, aux_ref, idx_ref, w_ref, acc_ref):
    i = pl.program_id(0)
    nb = pl.num_programs(0)

    @pl.when(i == 0)
    def _():
        acc_ref[0] = 0.0
        acc_ref[1] = 0.0
        acc_ref[2] = 0.0

    gr = jnp.transpose(gT[...])
    cr = jnp.concatenate(
        [jnp.transpose(cTa[...]), jnp.transpose(cTb[...])], axis=1)
    gm = jnp.mean(gr, axis=1, keepdims=True)
    gv = jnp.mean((gr - gm) ** 2, axis=1, keepdims=True)
    gln = (gr - gm) / jnp.sqrt(gv + 1e-5) * lngw[...] + lngb[...]
    cm = jnp.mean(cr, axis=1, keepdims=True)
    cv = jnp.mean((cr - cm) ** 2, axis=1, keepdims=True)
    cln = (cr - cm) / jnp.sqrt(cv + 1e-5) * lncw[...] + lncb[...]

    g = _mm(gln, pgw[...]) + pgb[...]
    c = _mm(cln, pcw[...]) + pcb[...]
    rp = _mm(g, rpw[...]) + rpb[...]
    rs = _mm(g, rsw[...]) + rsb[...]

    concat = jnp.concatenate([rp, c], axis=1)
    gh = jnp.maximum(_mm(concat, g1w[...]) + g1b[...], 0.0)
    gl0 = jnp.sum(gh * g2w[0:1, :], axis=1, keepdims=True) + g2b[:, 0:1]
    gl1 = jnp.sum(gh * g2w[1:2, :], axis=1, keepdims=True) + g2b[:, 1:2]
    m = jnp.maximum(gl0, gl1)
    e0 = jnp.exp(gl0 - m)
    e1 = jnp.exp(gl1 - m)
    s = e0 + e1
    gp0 = e0 / s
    gp1 = e1 / s

    fused_ref[...] = jnp.concatenate([rp * gp0, c * gp1], axis=1)

    ent = -(gp0 * jnp.log(gp0 + 1e-8) + gp1 * jnp.log(gp1 + 1e-8))
    acc_ref[0] += jnp.sum(ent)

    rpn = rp / jnp.maximum(jnp.sqrt(jnp.sum(rp * rp, axis=1, keepdims=True)), 1e-12)
    rsn = rs / jnp.maximum(jnp.sqrt(jnp.sum(rs * rs, axis=1, keepdims=True)), 1e-12)
    acc_ref[1] += jnp.sum(jnp.abs(jnp.sum(rpn * rsn, axis=1)))

    acc_ref[2] += jnp.sum((rs - c) ** 2)

    @pl.when(i == nb - 1)
    def _():
        aux_ref[0, 0] = (0.01 * (acc_ref[0] / _B)
                         + 0.02 * (acc_ref[1] / _B)
                         + 0.01 * (acc_ref[2] / (_B * _HID)))

    xj = xjr[...]
    yj = yjr[...]
    mj = mjr[...]
    xi = jnp.transpose(xjr[:, pl.ds(i * _R1, _R1)])
    yi = jnp.transpose(yjr[:, pl.ds(i * _R1, _R1)])
    mi = jnp.transpose(mjr[:, pl.ds(i * _R1, _R1)])

    dx = xi - xj
    dy = yi - yj
    d2 = dx * dx + dy * dy

    _SENT = 4e18
    _GONE = 1e19

    rowid = i * _R1 + lax.broadcasted_iota(jnp.int32, (_R1, 1), 0)
    colid = lax.broadcasted_iota(jnp.int32, (_R1, _B), 1)
    eye = colid == rowid
    same = (mi == mj) & jnp.logical_not(eye)
    has = jnp.any(same, axis=1, keepdims=True)
    d2 = jnp.where(has, jnp.where(same, d2, _SENT), d2)
    d2 = jnp.where(eye, _SENT, d2)

    work = d2
    vals = []
    idxs = []
    for k in range(_K):
        mn = jnp.min(work, axis=1, keepdims=True)
        am = jnp.min(jnp.where(work == mn, colid, _B), axis=1, keepdims=True)
        vals.append(mn)
        idxs.append(am)
        if k < _K - 1:
            work = jnp.where(colid == am, _GONE, work)

    v8 = jnp.concatenate(vals, axis=1)
    idx_ref[...] = jnp.concatenate(idxs, axis=1)
    s8 = jnp.where(v8 >= _SENT * 0.5, -1e9, -jnp.sqrt(v8 + 1e-12))
    vm = jnp.max(s8, axis=1, keepdims=True)
    ev = jnp.exp(s8 - vm)
    w = ev / jnp.sum(ev, axis=1, keepdims=True)
    w_ref[...] = jnp.concatenate([w, jnp.zeros((_R1, 16 - _K), jnp.float32)],
                                 axis=1)


def _p4_body(agg_ref, fused_ref, g1w, g1b, g2w, g2b, c1w, c1b,
             bnw, bnb, bnm, bnv, c2w, c2b, out_ref):
    h1 = jnp.maximum(_mm(agg_ref[...], g1w[...]) + g1b[...], 0.0)
    gu = _mm(h1, g2w[...]) + g2b[...]
    ctx = fused_ref[...] + _ALPHA * gu
    h = _mm(ctx, c1w[...]) + c1b[...]
    h = (h - bnm[...]) / jnp.sqrt(bnv[...] + 1e-5) * bnw[...] + bnb[...]
    h = jnp.maximum(h, 0.0)
    out_ref[...] = _mm(h, c2w[...]) + c2b[...]


_CHK = _CH * _K
_NCH = _BPW // _CH


def _sc_body(fused_hbm, idx_hbm, w_hbm, out_hbm,
             idx_v, w0, w1, rows0, rows1, out0, out1, sem0, sem1):
    wid = lax.axis_index("s") * 2 + lax.axis_index("c")
    base = wid * _BPW

    pltpu.sync_copy(idx_hbm.at[pl.ds(base * _K, _BPW * _K)], idx_v)

    rows = (rows0, rows1)
    wbufs = (w0, w1)
    outs = (out0, out1)
    sems = (sem0, sem1)

    def gidx(ci):
        return idx_v.at[pl.ds(ci * _CHK, _CHK)]

    def wsrc(ci):
        return w_hbm.at[pl.ds(base + ci * _CH, _CH)]

    def stage(ci, b):
        pltpu.async_copy(wsrc(ci), wbufs[b], sems[b])
        pltpu.async_copy(fused_hbm.at[gidx(ci)], rows[b], sems[b])

    def compute(ci, b):
        pltpu.make_async_copy(wsrc(ci), wbufs[b], sems[b]).wait()
        pltpu.make_async_copy(fused_hbm.at[gidx(ci)], rows[b], sems[b]).wait()
        rv = rows[b]
        ov = outs[b]
        for r in range(_CH):
            wrow = wbufs[b][r]
            dn = lax.GatherDimensionNumbers(
                offset_dims=(), collapsed_slice_dims=(0,),
                start_index_map=(0,))
            wv = [lax.gather(wrow, jnp.full((16, 1), k, jnp.int32), dn, (1,),
                             mode=lax.GatherScatterMode.PROMISE_IN_BOUNDS)
                  for k in range(_K)]

            def colbody(cc, cr2):
                for j in range(4):
                    sl = pl.ds(cc * 64 + j * 16, 16)
                    acc = rv[r * _K, sl] * wv[0]
                    for k in range(1, _K):
                        acc = acc + rv[r * _K + k, sl] * wv[k]
                    ov[r, sl] = acc
                return cr2

            lax.fori_loop(0, _DC // 64, colbody, 0)
        pltpu.sync_copy(ov, out_hbm.at[pl.ds(base + ci * _CH, _CH)])

    stage(0, 0)

    def lbody(j, carry):
        ci0 = 2 * j
        stage(ci0 + 1, 1)
        compute(ci0, 0)
        stage(ci0 + 2, 0)
        compute(ci0 + 1, 1)
        return carry

    lax.fori_loop(0, _NCH // 2 - 1, lbody, 0)
    stage(_NCH - 1, 1)
    compute(_NCH - 2, 0)
    compute(_NCH - 1, 1)


def _sc_aggregate(fused, idx_flat, w_exp):
    mesh = plsc.VectorSubcoreMesh(core_axis_name="c", subcore_axis_name="s")
    return pl.kernel(
        _sc_body,
        out_type=jax.ShapeDtypeStruct((_B, _DC), jnp.float32),
        mesh=mesh,
        scratch_types=[
            pltpu.VMEM((_BPW * _K,), jnp.int32),
            pltpu.VMEM((_CH, 16), jnp.float32),
            pltpu.VMEM((_CH, 16), jnp.float32),
            pltpu.VMEM((_CHK, _DC), jnp.float32),
            pltpu.VMEM((_CHK, _DC), jnp.float32),
            pltpu.VMEM((_CH, _DC), jnp.float32),
            pltpu.VMEM((_CH, _DC), jnp.float32),
            pltpu.SemaphoreType.DMA,
            pltpu.SemaphoreType.DMA,
        ],
    )(fused, idx_flat, w_exp)


def kernel(x, ln_g_w, ln_g_b, ln_c_w, ln_c_b, proj_g_W, proj_g_b, proj_c_W,
           proj_c_b, relp_W, relp_b, rels_W, rels_b, gate1_W, gate1_b,
           gate2_W, gate2_b, gu1_W, gu1_b, gu2_W, gu2_b, cls1_W, cls1_b,
           bn_w, bn_b, bn_mean, bn_var, cls2_W, cls2_b):
    xT = x.T
    xrow = lax.slice(xT, (_DG + _DC, 0), (_DG + _DC + 1, _B))
    yrow = lax.slice(xT, (_DG + _DC + 1, 0), (_DG + _DC + 2, _B))
    mrow = lax.slice(xT, (_DG + _DC + 4, 0), (_DG + _DC + 5, _B))

    row = lambda v: v.reshape(1, -1)
    full = lambda shape: pl.BlockSpec(shape, lambda i: (0, 0))

    nb1 = _B // _R1
    fused, aux, idx8, w8 = pl.pallas_call(
        _p1_body,
        grid=(nb1,),
        in_specs=[
            pl.BlockSpec((_DG, _R1), lambda i: (0, i)),
            pl.BlockSpec((_DG, _R1), lambda i: (1, i)),
            pl.BlockSpec((_DG, _R1), lambda i: (2, i)),
            full((1, _B)),
            full((1, _B)),
            full((1, _B)),
            full((1, _DG)), full((1, _DG)), full((1, _DC)), full((1, _DC)),
            full((_HID, _DG)), full((1, _HID)),
            full((_HID, _DC)), full((1, _HID)),
            full((_HID, _HID)), full((1, _HID)),
            full((_HID, _HID)), full((1, _HID)),
            full((_GH, 2 * _HID)), full((1, _GH)),
            full((2, _GH)), full((1, 2)),
        ],
        out_specs=[
            pl.BlockSpec((_R1, _DC), lambda i: (i, 0)),
            pl.BlockSpec(memory_space=pltpu.SMEM),
            pl.BlockSpec((_R1, _K), lambda i: (i, 0)),
            pl.BlockSpec((_R1, 16), lambda i: (i, 0)),
        ],
        out_shape=[
            jax.ShapeDtypeStruct((_B, _DC), jnp.float32),
            jax.ShapeDtypeStruct((1, 1), jnp.float32),
            jax.ShapeDtypeStruct((_B, _K), jnp.int32),
            jax.ShapeDtypeStruct((_B, 16), jnp.float32),
        ],
        scratch_shapes=[pltpu.SMEM((3,), jnp.float32)],
    )(xT, xT, xT, xrow, yrow, mrow, row(ln_g_w), row(ln_g_b), row(ln_c_w), row(ln_c_b),
      proj_g_W, row(proj_g_b), proj_c_W, row(proj_c_b),
      relp_W, row(relp_b), rels_W, row(rels_b),
      gate1_W, row(gate1_b), gate2_W, row(gate2_b))

    agg = _sc_aggregate(fused, idx8.reshape(-1), w8)

    nb4 = _B // _R4
    logits = pl.pallas_call(
        _p4_body,
        grid=(nb4,),
        in_specs=[
            pl.BlockSpec((_R4, _DC), lambda i: (i, 0)),
            pl.BlockSpec((_R4, _DC), lambda i: (i, 0)),
            full((2 * _HID, 2 * _HID)), full((1, 2 * _HID)),
            full((2 * _HID, 2 * _HID)), full((1, 2 * _HID)),
            full((_HID, 2 * _HID)), full((1, _HID)),
            full((1, _HID)), full((1, _HID)), full((1, _HID)), full((1, _HID)),
            full((_NC, _HID)), full((1, _NC)),
        ],
        out_specs=pl.BlockSpec((_R4, _NC), lambda i: (i, 0)),
        out_shape=jax.ShapeDtypeStruct((_B, _NC), jnp.float32),
    )(agg, fused, gu1_W, row(gu1_b), gu2_W, row(gu2_b),
      cls1_W, row(cls1_b), row(bn_w), row(bn_b), row(bn_mean), row(bn_var),
      cls2_W, row(cls2_b))

    return logits, aux[0, 0]
